# baseline (speedup 1.0000x reference)
import jax
import jax.numpy as jnp
from jax import lax
from jax.experimental import pallas as pl
from jax.experimental.pallas import tpu as pltpu
from jax.experimental.pallas import tpu_sc as plsc

_N = 50000
_E = 500000
_D = 128
_NQ = 8
_DQ = _D // _NQ
_SUB = 128
_NS = 16
_GRP = 8
_NG = 31
_ROWS = _GRP * _NG
_EPAD = _NS * _ROWS * _SUB
_NPAD = 50048
_RPT = _NPAD // _NS


def _seg_side(s, tbls, idx_r, outs, out_w, zeros_r, drain_r, ibuf, rows, acc,
              sem_g, sem_s, sem_i):
  base = s * _RPT
  pltpu.sync_copy(zeros_r, acc.at[pl.ds(base, _RPT)])

  for q in range(_NQ):
    plsc.subcore_barrier()
    tbl = tbls[q]

    def idx_desc(g, sl):
      return pltpu.make_async_copy(idx_r.at[s, g], ibuf.at[sl], sem_i)

    def gather_desc(p, sl, j):
      return pltpu.make_async_copy(tbl.at[ibuf.at[sl, 0].at[j]],
                                   rows.at[p, j], sem_g)

    def scatter_desc(p, sl, j):
      return pltpu.make_async_copy(rows.at[p, j],
                                   acc.at[ibuf.at[sl, 1].at[j]], sem_s)

    idx_desc(0, 0).start()
    idx_desc(1, 1).start()

    def pipe(gg, carry):
      for u in range(4):
        g = gg * 4 + u
        p = u % 2
        sl = u
        sl1 = (u - 1) % 4
        sl2 = (u + 2) % 4

        @pl.when(jnp.logical_and(g >= 2, g < _NG + 2))
        def _():
          pltpu.make_async_copy(drain_r, rows.at[p], sem_s).wait()

        @pl.when(g + 2 < _NG)
        def _():
          idx_desc(g + 2, sl2).start()

        @pl.when(g < _NG)
        def _():
          idx_desc(g, sl).wait()
          for j in range(_GRP):
            gather_desc(p, sl, j).start()

        @pl.when(jnp.logical_and(g >= 1, g < _NG + 1))
        def _():
          pltpu.make_async_copy(drain_r, rows.at[1 - p], sem_g).wait()
          for j in range(_GRP):
            scatter_desc(1 - p, sl1, j).start(add=True)

      return carry

    lax.fori_loop(0, (_NG + 2 + 3) // 4, pipe, 0)

    plsc.subcore_barrier()
    pltpu.sync_copy(acc.at[pl.ds(base, _RPT)], outs[q].at[pl.ds(base, _RPT)])
    pltpu.sync_copy(acc.at[pl.ds(base, _RPT)],
                    out_w.at[pl.ds(base, _RPT), pl.ds(q * _DQ, _DQ)])
    if q < _NQ - 1:
      pltpu.sync_copy(zeros_r, acc.at[pl.ds(base, _RPT)])


def _sc_body(*refs):
  tu = refs[0:_NQ]
  ti = refs[_NQ:2 * _NQ]
  idx_u, idx_i, zeros_r, drain_r = refs[2 * _NQ:2 * _NQ + 4]
  ou = refs[2 * _NQ + 4:3 * _NQ + 4]
  oi = refs[3 * _NQ + 4:4 * _NQ + 4]
  ou_w, oi_w = refs[4 * _NQ + 4:4 * _NQ + 6]
  ibuf, rows, acc, sem_g, sem_s, sem_i = refs[4 * _NQ + 6:]
  c = lax.axis_index("c")
  s = lax.axis_index("s")

  @pl.when(c == 0)
  def _():
    _seg_side(s, tu, idx_u, ou, ou_w, zeros_r, drain_r, ibuf, rows, acc,
              sem_g, sem_s, sem_i)

  @pl.when(c == 1)
  def _():
    _seg_side(s, ti, idx_i, oi, oi_w, zeros_r, drain_r, ibuf, rows, acc,
              sem_g, sem_s, sem_i)


def _make_sc_call():
  f32 = jnp.float32
  return pl.kernel(
      _sc_body,
      out_type=([jax.ShapeDtypeStruct((_NPAD, _DQ), f32)] * (2 * _NQ) +
                [jax.ShapeDtypeStruct((_NPAD, _D), f32)] * 2),
      mesh=plsc.VectorSubcoreMesh(core_axis_name="c", subcore_axis_name="s"),
      scratch_types=[
          pltpu.VMEM((4, 2, _GRP, _SUB), jnp.int32),
          pltpu.VMEM((2, _GRP, _SUB, _DQ), f32),
          pltpu.VMEM_SHARED((_NPAD, _DQ), f32),
          pltpu.SemaphoreType.DMA,
          pltpu.SemaphoreType.DMA,
          pltpu.SemaphoreType.DMA,
      ],
      compiler_params=pltpu.CompilerParams(use_tc_tiling_on_sc=False),
  )


_R = 1000


def _tap0_body(xu, xi, w0u, w0i, bu, bi, zu, zi):
  zu[...] = (jnp.dot(xu[...], w0u[...], preferred_element_type=jnp.float32)
             + jnp.sum(bu[...], axis=0, keepdims=True))
  zi[...] = (jnp.dot(xi[...], w0i[...], preferred_element_type=jnp.float32)
             + jnp.sum(bi[...], axis=0, keepdims=True))


def _tapn_body(zu0, zi0, mu, mi, wu, wi, zu, zi):
  zu[...] = zu0[...] + jnp.dot(mu[...], wu[...],
                               preferred_element_type=jnp.float32)
  zi[...] = zi0[...] + jnp.dot(mi[...], wi[...],
                               preferred_element_type=jnp.float32)


def _make_tap0_call():
  f32 = jnp.float32
  blk = lambda shape: pl.BlockSpec(shape, lambda i: (i, 0))
  rep = lambda shape: pl.BlockSpec(shape, lambda i: (0, 0))
  return pl.pallas_call(
      _tap0_body,
      grid=(_N // _R,),
      in_specs=[blk((_R, _D))] * 2 + [rep((_D, _D))] * 2 + [rep((3, _D))] * 2,
      out_specs=[blk((_R, _D))] * 2,
      out_shape=[jax.ShapeDtypeStruct((_N, _D), f32)] * 2,
  )


def _make_tapn_call():
  f32 = jnp.float32
  blk = lambda shape: pl.BlockSpec(shape, lambda i: (i, 0))
  rep = lambda shape: pl.BlockSpec(shape, lambda i: (0, 0))
  return pl.pallas_call(
      _tapn_body,
      grid=(_N // _R,),
      in_specs=[blk((_R, _D))] * 4 + [rep((_D, _D))] * 2,
      out_specs=[blk((_R, _D))] * 2,
      out_shape=[jax.ShapeDtypeStruct((_N, _D), f32)] * 2,
  )


def _pack_edges(ei):
  i32 = jnp.int32
  src = ei[0].astype(i32)
  dst = ei[1].astype(i32)
  pad = _EPAD - _E
  src = jnp.concatenate([src, jnp.zeros((pad,), i32)])
  dst = jnp.concatenate([dst, jnp.full((pad,), _NPAD - 1, i32)])
  packed = jnp.stack([src, dst]).reshape(2, _NS, _NG, _GRP, _SUB)
  return packed.transpose(1, 2, 0, 3, 4)


def kernel(x_user, x_item, ei_user_to_item, ei_item_to_user,
           W0_user, b0_user, W0_item, b0_item,
           W1_user, b1_user, W1_item, b1_item,
           W2_user, b2_user, W2_item, b2_item):
  idx_u = _pack_edges(ei_item_to_user)
  idx_i = _pack_edges(ei_user_to_item)
  tu = [x_item[:, q * _DQ:(q + 1) * _DQ] for q in range(_NQ)]
  ti = [x_user[:, q * _DQ:(q + 1) * _DQ] for q in range(_NQ)]
  zeros = jnp.zeros((_RPT, _DQ), jnp.float32)
  drain = jnp.zeros((_GRP, _SUB, _DQ), jnp.float32)

  bu = jnp.stack([b0_user, b1_user, b2_user])
  bi = jnp.stack([b0_item, b1_item, b2_item])

  sc = _make_sc_call()
  o1 = sc(*tu, *ti, idx_u, idx_i, zeros, drain)
  mu1, mi1 = o1[:_NQ], o1[_NQ:2 * _NQ]
  mu1_w, mi1_w = o1[2 * _NQ], o1[2 * _NQ + 1]
  z0u, z0i = _make_tap0_call()(x_user, x_item, W0_user, W0_item, bu, bi)
  o2 = sc(*mi1, *mu1, idx_u, idx_i, zeros, drain)
  mu2_w, mi2_w = o2[2 * _NQ], o2[2 * _NQ + 1]
  tapn = _make_tapn_call()
  z1u, z1i = tapn(z0u, z0i, mu1_w, mi1_w, W1_user, W1_item)
  z_user, z_item = tapn(z1u, z1i, mu2_w, mi2_w, W2_user, W2_item)
  return (z_user, z_item)


# speedup vs baseline: 1.0219x; 1.0219x over previous
import jax
import jax.numpy as jnp
from jax import lax
from jax.experimental import pallas as pl
from jax.experimental.pallas import tpu as pltpu
from jax.experimental.pallas import tpu_sc as plsc

_N = 50000
_E = 500000
_D = 128
_NQ = 8
_DQ = _D // _NQ
_SUB = 128
_NS = 16
_GRP = 8
_NG = 31
_ROWS = _GRP * _NG
_EPAD = _NS * _ROWS * _SUB
_NPAD = 50048
_RPT = _NPAD // _NS


def _seg_side(s, tbls, idx_r, outs, out_w, zeros_r, drain_r, ibuf, rows, acc,
              sem_g, sem_s, sem_i):
  base = s * _RPT
  pltpu.sync_copy(zeros_r, acc.at[pl.ds(base, _RPT)])

  for q in range(_NQ):
    plsc.subcore_barrier()
    tbl = tbls[q]

    def idx_desc(g, sl):
      return pltpu.make_async_copy(idx_r.at[s, g], ibuf.at[sl], sem_i)

    def gather_desc(p, sl, j):
      return pltpu.make_async_copy(tbl.at[ibuf.at[sl, 0].at[j]],
                                   rows.at[p, j], sem_g)

    def scatter_desc(p, sl, j):
      return pltpu.make_async_copy(rows.at[p, j],
                                   acc.at[ibuf.at[sl, 1].at[j]], sem_s)

    idx_desc(0, 0).start()
    idx_desc(1, 1).start()

    def pipe(gg, carry):
      for u in range(4):
        g = gg * 4 + u
        p = u % 2
        sl = u
        sl1 = (u - 1) % 4
        sl2 = (u + 2) % 4

        @pl.when(jnp.logical_and(g >= 2, g < _NG + 2))
        def _():
          pltpu.make_async_copy(drain_r, rows.at[p], sem_s).wait()

        @pl.when(g + 2 < _NG)
        def _():
          idx_desc(g + 2, sl2).start()

        @pl.when(g < _NG)
        def _():
          idx_desc(g, sl).wait()
          for j in range(_GRP):
            gather_desc(p, sl, j).start()

        @pl.when(jnp.logical_and(g >= 1, g < _NG + 1))
        def _():
          pltpu.make_async_copy(drain_r, rows.at[1 - p], sem_g).wait()
          for j in range(_GRP):
            scatter_desc(1 - p, sl1, j).start(add=True)

      return carry

    lax.fori_loop(0, (_NG + 2 + 3) // 4, pipe, 0)

    plsc.subcore_barrier()
    pltpu.sync_copy(acc.at[pl.ds(base, _RPT)], outs[q].at[pl.ds(base, _RPT)])
    pltpu.sync_copy(acc.at[pl.ds(base, _RPT)],
                    out_w.at[pl.ds(base, _RPT), pl.ds(q * _DQ, _DQ)])
    if q < _NQ - 1:
      pltpu.sync_copy(zeros_r, acc.at[pl.ds(base, _RPT)])


def _sc_body(*refs):
  tu = refs[0:_NQ]
  ti = refs[_NQ:2 * _NQ]
  idx_u, idx_i, zeros_r, drain_r = refs[2 * _NQ:2 * _NQ + 4]
  ou = refs[2 * _NQ + 4:3 * _NQ + 4]
  oi = refs[3 * _NQ + 4:4 * _NQ + 4]
  ou_w, oi_w = refs[4 * _NQ + 4:4 * _NQ + 6]
  ibuf, rows, acc, sem_g, sem_s, sem_i = refs[4 * _NQ + 6:]
  c = lax.axis_index("c")
  s = lax.axis_index("s")

  @pl.when(c == 0)
  def _():
    _seg_side(s, tu, idx_u, ou, ou_w, zeros_r, drain_r, ibuf, rows, acc,
              sem_g, sem_s, sem_i)

  @pl.when(c == 1)
  def _():
    _seg_side(s, ti, idx_i, oi, oi_w, zeros_r, drain_r, ibuf, rows, acc,
              sem_g, sem_s, sem_i)


def _make_sc_call():
  f32 = jnp.float32
  return pl.kernel(
      _sc_body,
      out_type=([jax.ShapeDtypeStruct((_NPAD, _DQ), f32)] * (2 * _NQ) +
                [jax.ShapeDtypeStruct((_NPAD, _D), f32)] * 2),
      mesh=plsc.VectorSubcoreMesh(core_axis_name="c", subcore_axis_name="s"),
      scratch_types=[
          pltpu.VMEM((4, 2, _GRP, _SUB), jnp.int32),
          pltpu.VMEM((2, _GRP, _SUB, _DQ), f32),
          pltpu.VMEM_SHARED((_NPAD, _DQ), f32),
          pltpu.SemaphoreType.DMA,
          pltpu.SemaphoreType.DMA,
          pltpu.SemaphoreType.DMA,
      ],
      compiler_params=pltpu.CompilerParams(use_tc_tiling_on_sc=False),
  )


_R = 1000


def _taps_body(xu, xi, mu1, mu2, mi1, mi2,
               w0u, w1u, w2u, w0i, w1i, w2i, bu, bi, zu, zi):
  def side(x, m1, m2, w0, w1, w2, b, z):
    acc = jnp.dot(x[...], w0[...], preferred_element_type=jnp.float32)
    acc = acc + jnp.dot(m1[...], w1[...], preferred_element_type=jnp.float32)
    acc = acc + jnp.dot(m2[...], w2[...], preferred_element_type=jnp.float32)
    z[...] = acc + jnp.sum(b[...], axis=0, keepdims=True)

  side(xu, mu1, mu2, w0u, w1u, w2u, bu, zu)
  side(xi, mi1, mi2, w0i, w1i, w2i, bi, zi)


def _make_tc_call():
  f32 = jnp.float32
  blk = lambda shape: pl.BlockSpec(shape, lambda i: (i, 0))
  rep = lambda shape: pl.BlockSpec(shape, lambda i: (0, 0))
  in_specs = ([blk((_R, _D))] * 6 +
              [rep((_D, _D))] * 6 + [rep((3, _D))] * 2)
  return pl.pallas_call(
      _taps_body,
      grid=(_N // _R,),
      in_specs=in_specs,
      out_specs=[blk((_R, _D))] * 2,
      out_shape=[jax.ShapeDtypeStruct((_N, _D), f32)] * 2,
  )


def _pack_edges(ei):
  i32 = jnp.int32
  src = ei[0].astype(i32)
  dst = ei[1].astype(i32)
  pad = _EPAD - _E
  src = jnp.concatenate([src, jnp.zeros((pad,), i32)])
  dst = jnp.concatenate([dst, jnp.full((pad,), _NPAD - 1, i32)])
  packed = jnp.stack([src, dst]).reshape(2, _NS, _NG, _GRP, _SUB)
  return packed.transpose(1, 2, 0, 3, 4)


def kernel(x_user, x_item, ei_user_to_item, ei_item_to_user,
           W0_user, b0_user, W0_item, b0_item,
           W1_user, b1_user, W1_item, b1_item,
           W2_user, b2_user, W2_item, b2_item):
  idx_u = _pack_edges(ei_item_to_user)
  idx_i = _pack_edges(ei_user_to_item)
  tu = [x_item[:, q * _DQ:(q + 1) * _DQ] for q in range(_NQ)]
  ti = [x_user[:, q * _DQ:(q + 1) * _DQ] for q in range(_NQ)]
  zeros = jnp.zeros((_RPT, _DQ), jnp.float32)
  drain = jnp.zeros((_GRP, _SUB, _DQ), jnp.float32)

  sc = _make_sc_call()
  o1 = sc(*tu, *ti, idx_u, idx_i, zeros, drain)
  mu1, mi1 = o1[:_NQ], o1[_NQ:2 * _NQ]
  mu1_w, mi1_w = o1[2 * _NQ], o1[2 * _NQ + 1]
  o2 = sc(*mi1, *mu1, idx_u, idx_i, zeros, drain)
  mu2_w, mi2_w = o2[2 * _NQ], o2[2 * _NQ + 1]

  bu = jnp.stack([b0_user, b1_user, b2_user])
  bi = jnp.stack([b0_item, b1_item, b2_item])
  z_user, z_item = _make_tc_call()(
      x_user, x_item, mu1_w, mu2_w, mi1_w, mi2_w,
      W0_user, W1_user, W2_user, W0_item, W1_item, W2_item, bu, bi)
  return (z_user, z_item)
